# baseline (speedup 1.0000x reference)
import functools

import jax
import jax.numpy as jnp
from jax import lax
from jax.experimental import pallas as pl
from jax.experimental.pallas import tpu as pltpu
from jax.experimental.pallas import tpu_sc as plsc

N = 100000
M = 10000
E = 160000
K = 27
H = 128
KM = K * M

NC = 2
NS = 16
CH = 80
NCH_P = N // CH
NCH_E = E // CH
MT0 = 624
MTL = M - (NS - 1) * MT0

_mesh = plsc.VectorSubcoreMesh(
    core_axis_name="c", subcore_axis_name="s", num_cores=NC, num_subcores=NS)

_f32 = jnp.float32


def _cs():
    return lax.axis_index("c"), lax.axis_index("s")


def _tile_copy(s, src_fn, dst_fn):
    base = pl.multiple_of(s * MT0, 8)
    last = (NS - 1) * MT0

    @pl.when(s < NS - 1)
    def _():
        pltpu.sync_copy(src_fn(base, MT0), dst_fn(base, MT0))

    @pl.when(s == NS - 1)
    def _():
        pltpu.sync_copy(src_fn(last, MTL), dst_fn(last, MTL))


NB = 4


def _unpack_idx(all_v, blk_v, delta, nb=NB):
    for b in range(nb):
        for u in range(CH // 16):
            v = all_v[pl.ds(b * CH + u * 16, 16)]
            if delta is not None:
                v = v + delta
            blk_v[b, pl.ds(u * 16, 16)] = v


def _tile_span(total, s):
    q0, rem = divmod(total, NS)
    base = s * q0 + jnp.minimum(s, rem)
    n = q0 + jnp.where(s < rem, 1, 0)
    return base, n


@functools.partial(
    pl.kernel,
    out_type=jax.ShapeDtypeStruct((NC * M, H), _f32),
    mesh=_mesh,
    scratch_types=[
        pltpu.VMEM_SHARED((M, H), _f32),
        pltpu.VMEM((NB * CH,), jnp.int32),
        pltpu.VMEM((NB, CH), jnp.int32),
        pltpu.VMEM((NB, CH, H), _f32),
        pltpu.SemaphoreType.DMA,
        pltpu.SemaphoreType.DMA,
    ],
)
def _sc_vox_ps(data_hbm, p2v_hbm, z128,
               sums_o, acc, all_v, blk_v, rows_v, sem, sem2):
    c, s = _cs()
    _tile_copy(s, lambda b, n: z128.at[pl.ds(0, n)],
               lambda b, n: acc.at[pl.ds(b, n)])
    plsc.subcore_barrier()

    half = NCH_P // NC
    base, n_t = _tile_span(half, s)
    base = base + c * half
    nr = n_t // NB

    def rnd(r, carry):
        cb = base + r * NB
        off = pl.multiple_of(cb * CH, 8)
        pltpu.sync_copy(p2v_hbm.at[pl.ds(off, NB * CH)], all_v)
        _unpack_idx(all_v, blk_v, None)
        gd = [pltpu.async_copy(
            data_hbm.at[pl.ds(pl.multiple_of((cb + b) * CH, 8), CH)],
            rows_v.at[b], sem) for b in range(NB)]
        for d in gd:
            d.wait()
        sd = [pltpu.async_copy(rows_v.at[b], acc.at[blk_v.at[b]], sem2,
                               add=True) for b in range(NB)]
        for d in sd:
            d.wait()
        return carry
    lax.fori_loop(0, nr, rnd, 0)

    def tail(i, carry):
        cb = base + nr * NB + i
        off = pl.multiple_of(cb * CH, 8)
        pltpu.sync_copy(p2v_hbm.at[pl.ds(off, CH)], all_v.at[pl.ds(0, CH)])
        _unpack_idx(all_v, blk_v, None, nb=1)
        pltpu.sync_copy(data_hbm.at[pl.ds(off, CH)], rows_v.at[0])
        pltpu.sync_copy(rows_v.at[0], acc.at[blk_v.at[0]], add=True)
        return carry
    lax.fori_loop(0, n_t - nr * NB, tail, 0)

    plsc.subcore_barrier()
    cm = pl.multiple_of(c * M, 8)
    _tile_copy(s, lambda b, n: acc.at[pl.ds(b, n)],
               lambda b, n: sums_o.at[pl.ds(cm + b, n)])


@functools.partial(
    pl.kernel,
    out_type=jax.ShapeDtypeStruct((NC * M, H), _f32),
    mesh=_mesh,
    scratch_types=[
        pltpu.VMEM_SHARED((M, H), _f32),
        pltpu.VMEM((NB * CH,), jnp.int32),
        pltpu.VMEM((NB, CH), jnp.int32),
        pltpu.VMEM((CH, H), _f32),
        pltpu.SemaphoreType.DMA,
    ],
)
def _sc_cnt_ps(p2v_hbm, z128, ones128, cnt_o, acc, all_v, blk_v, ones_v,
               sem2):
    c, s = _cs()
    _tile_copy(s, lambda b, n: z128.at[pl.ds(0, n)],
               lambda b, n: acc.at[pl.ds(b, n)])
    pltpu.sync_copy(ones128, ones_v)
    plsc.subcore_barrier()

    half = NCH_P // NC
    base, n_t = _tile_span(half, s)
    base = base + c * half
    nr = n_t // NB

    def rnd(r, carry):
        cb = base + r * NB
        off = pl.multiple_of(cb * CH, 8)
        pltpu.sync_copy(p2v_hbm.at[pl.ds(off, NB * CH)], all_v)
        _unpack_idx(all_v, blk_v, None)
        sd = [pltpu.async_copy(ones_v, acc.at[blk_v.at[b]], sem2,
                               add=True) for b in range(NB)]
        for d in sd:
            d.wait()
        return carry
    lax.fori_loop(0, nr, rnd, 0)

    def tail(i, carry):
        cb = base + nr * NB + i
        off = pl.multiple_of(cb * CH, 8)
        pltpu.sync_copy(p2v_hbm.at[pl.ds(off, CH)], all_v.at[pl.ds(0, CH)])
        _unpack_idx(all_v, blk_v, None, nb=1)
        pltpu.sync_copy(ones_v, acc.at[blk_v.at[0]], add=True)
        return carry
    lax.fori_loop(0, n_t - nr * NB, tail, 0)

    plsc.subcore_barrier()
    cm = pl.multiple_of(c * M, 8)
    _tile_copy(s, lambda b, n: acc.at[pl.ds(b, n)],
               lambda b, n: cnt_o.at[pl.ds(cm + b, n)])


@functools.partial(
    pl.kernel,
    out_type=jax.ShapeDtypeStruct((NC * M, H), _f32),
    mesh=_mesh,
    scratch_types=[
        pltpu.VMEM_SHARED((M, H), _f32),
        pltpu.VMEM((NB * CH,), jnp.int32),
        pltpu.VMEM((NB * CH,), jnp.int32),
        pltpu.VMEM((NB, CH), jnp.int32),
        pltpu.VMEM((NB, CH), jnp.int32),
        pltpu.VMEM((NB, CH, H), _f32),
        pltpu.SemaphoreType.DMA,
        pltpu.SemaphoreType.DMA,
    ],
)
def _sc_edge_zr(tv2_hbm, gidx_hbm, dst_hbm, z128, out2_o,
                acc, gi_all, di_all, gi_blk, di_blk, rows_v, sem, sem2):
    c, s = _cs()
    _tile_copy(s, lambda b, n: z128.at[pl.ds(0, n)],
               lambda b, n: acc.at[pl.ds(b, n)])
    plsc.subcore_barrier()

    base, n_t = _tile_span(NCH_E, s)
    nr = n_t // NB
    delta = c * KM

    def rnd(r, carry):
        cb = base + r * NB
        off = pl.multiple_of(cb * CH, 8)
        pltpu.sync_copy(gidx_hbm.at[pl.ds(off, NB * CH)], gi_all)
        pltpu.sync_copy(dst_hbm.at[pl.ds(off, NB * CH)], di_all)
        _unpack_idx(gi_all, gi_blk, delta)
        _unpack_idx(di_all, di_blk, None)
        gd = [pltpu.async_copy(tv2_hbm.at[gi_blk.at[b]], rows_v.at[b], sem)
              for b in range(NB)]
        for d in gd:
            d.wait()
        sd = [pltpu.async_copy(rows_v.at[b], acc.at[di_blk.at[b]], sem2,
                               add=True) for b in range(NB)]
        for d in sd:
            d.wait()
        return carry
    lax.fori_loop(0, nr, rnd, 0)

    def tail(i, carry):
        cb = base + nr * NB + i
        off = pl.multiple_of(cb * CH, 8)
        pltpu.sync_copy(gidx_hbm.at[pl.ds(off, CH)], gi_all.at[pl.ds(0, CH)])
        pltpu.sync_copy(dst_hbm.at[pl.ds(off, CH)], di_all.at[pl.ds(0, CH)])
        _unpack_idx(gi_all, gi_blk, delta, nb=1)
        _unpack_idx(di_all, di_blk, None, nb=1)
        pltpu.async_copy(tv2_hbm.at[gi_blk.at[0]], rows_v.at[0], sem).wait()
        pltpu.sync_copy(rows_v.at[0], acc.at[di_blk.at[0]], add=True)
        return carry
    lax.fori_loop(0, n_t - nr * NB, tail, 0)

    plsc.subcore_barrier()
    cm = pl.multiple_of(c * M, 8)
    _tile_copy(s, lambda b, n: acc.at[pl.ds(b, n)],
               lambda b, n: out2_o.at[pl.ds(cm + b, n)])


@functools.partial(
    pl.kernel,
    out_type=jax.ShapeDtypeStruct((NC * N, H), _f32),
    mesh=_mesh,
    scratch_types=[
        pltpu.VMEM((NB * CH,), jnp.int32),
        pltpu.VMEM((NB, CH), jnp.int32),
        pltpu.VMEM((NB, CH, H), _f32),
        pltpu.SemaphoreType.DMA,
        pltpu.SemaphoreType.DMA,
    ],
)
def _sc_devox_zr(ov2_hbm, p2v_hbm, g2_o, all_v, blk_v, rows_v, sem, sem2):
    c, s = _cs()
    base, n_t = _tile_span(NCH_P, s)
    nr = n_t // NB
    delta = c * M

    def rnd(r, carry):
        cb = base + r * NB
        off = pl.multiple_of(cb * CH, 8)
        pltpu.sync_copy(p2v_hbm.at[pl.ds(off, NB * CH)], all_v)
        _unpack_idx(all_v, blk_v, delta)
        gd = [pltpu.async_copy(ov2_hbm.at[blk_v.at[b]], rows_v.at[b], sem)
              for b in range(NB)]
        for d in gd:
            d.wait()
        cn = pl.multiple_of(c * N + off, 8)
        sd = [pltpu.async_copy(
            rows_v.at[b], g2_o.at[pl.ds(cn + b * CH, CH)], sem2)
            for b in range(NB)]
        for d in sd:
            d.wait()
        return carry
    lax.fori_loop(0, nr, rnd, 0)

    def tail(i, carry):
        cb = base + nr * NB + i
        off = pl.multiple_of(cb * CH, 8)
        pltpu.sync_copy(p2v_hbm.at[pl.ds(off, CH)], all_v.at[pl.ds(0, CH)])
        _unpack_idx(all_v, blk_v, delta, nb=1)
        pltpu.async_copy(ov2_hbm.at[blk_v.at[0]], rows_v.at[0], sem).wait()
        cn = pl.multiple_of(c * N + off, 8)
        pltpu.sync_copy(rows_v.at[0], g2_o.at[pl.ds(cn, CH)])
        return carry
    lax.fori_loop(0, n_t - nr * NB, tail, 0)


@functools.partial(
    pl.kernel,
    out_type=jax.ShapeDtypeStruct((NC * M, H), _f32),
    mesh=_mesh,
    scratch_types=[
        pltpu.VMEM_SHARED((M, H), _f32),
        pltpu.VMEM((NB * CH,), jnp.int32),
        pltpu.VMEM((NB * CH,), jnp.int32),
        pltpu.VMEM((NB, CH), jnp.int32),
        pltpu.VMEM((NB, CH), jnp.int32),
        pltpu.VMEM((NB, CH, H), _f32),
        pltpu.SemaphoreType.DMA,
        pltpu.SemaphoreType.DMA,
    ],
)
def _sc_edge_q(tvq_hbm, gidx_hbm, dst_hbm, z128, out2_o,
               acc, gi_all, di_all, gi_blk, di_blk, rows_v, sem, sem2):
    c, s = _cs()
    _tile_copy(s, lambda b, n: z128.at[pl.ds(0, n)],
               lambda b, n: acc.at[pl.ds(b, n)])
    plsc.subcore_barrier()

    half = NCH_E // NC
    base, n_t = _tile_span(half, s)
    base = base + c * half
    nr = n_t // NB

    def rnd(r, carry):
        cb = base + r * NB
        off = pl.multiple_of(cb * CH, 8)
        pltpu.sync_copy(gidx_hbm.at[pl.ds(off, NB * CH)], gi_all)
        pltpu.sync_copy(dst_hbm.at[pl.ds(off, NB * CH)], di_all)
        _unpack_idx(gi_all, gi_blk, None)
        _unpack_idx(di_all, di_blk, None)
        gd = [pltpu.async_copy(tvq_hbm.at[gi_blk.at[b]], rows_v.at[b], sem)
              for b in range(NB)]
        for d in gd:
            d.wait()
        sd = [pltpu.async_copy(rows_v.at[b], acc.at[di_blk.at[b]], sem2,
                               add=True) for b in range(NB)]
        for d in sd:
            d.wait()
        return carry
    lax.fori_loop(0, nr, rnd, 0)

    def tail(i, carry):
        cb = base + nr * NB + i
        off = pl.multiple_of(cb * CH, 8)
        pltpu.sync_copy(gidx_hbm.at[pl.ds(off, CH)], gi_all.at[pl.ds(0, CH)])
        pltpu.sync_copy(dst_hbm.at[pl.ds(off, CH)], di_all.at[pl.ds(0, CH)])
        _unpack_idx(gi_all, gi_blk, None, nb=1)
        _unpack_idx(di_all, di_blk, None, nb=1)
        pltpu.async_copy(tvq_hbm.at[gi_blk.at[0]], rows_v.at[0], sem).wait()
        pltpu.sync_copy(rows_v.at[0], acc.at[di_blk.at[0]], add=True)
        return carry
    lax.fori_loop(0, n_t - nr * NB, tail, 0)

    plsc.subcore_barrier()
    cm = pl.multiple_of(c * M, 8)
    _tile_copy(s, lambda b, n: acc.at[pl.ds(b, n)],
               lambda b, n: out2_o.at[pl.ds(cm + b, n)])


@functools.partial(
    pl.kernel,
    out_type=jax.ShapeDtypeStruct((N, H), _f32),
    mesh=_mesh,
    scratch_types=[
        pltpu.VMEM((NB * CH,), jnp.int32),
        pltpu.VMEM((NB, CH), jnp.int32),
        pltpu.VMEM((NB, CH, H), _f32),
        pltpu.SemaphoreType.DMA,
        pltpu.SemaphoreType.DMA,
    ],
)
def _sc_devox_q(tq_hbm, p2v_hbm, gq_o, all_v, blk_v, rows_v, sem, sem2):
    c, s = _cs()
    half = NCH_P // NC
    base, n_t = _tile_span(half, s)
    base = base + c * half
    nr = n_t // NB

    def rnd(r, carry):
        cb = base + r * NB
        off = pl.multiple_of(cb * CH, 8)
        pltpu.sync_copy(p2v_hbm.at[pl.ds(off, NB * CH)], all_v)
        _unpack_idx(all_v, blk_v, None)
        gd = [pltpu.async_copy(tq_hbm.at[blk_v.at[b]], rows_v.at[b], sem)
              for b in range(NB)]
        for d in gd:
            d.wait()
        sd = [pltpu.async_copy(
            rows_v.at[b], gq_o.at[pl.ds(pl.multiple_of(off + b * CH, 8), CH)],
            sem2) for b in range(NB)]
        for d in sd:
            d.wait()
        return carry
    lax.fori_loop(0, nr, rnd, 0)

    def tail(i, carry):
        cb = base + nr * NB + i
        off = pl.multiple_of(cb * CH, 8)
        pltpu.sync_copy(p2v_hbm.at[pl.ds(off, CH)], all_v.at[pl.ds(0, CH)])
        _unpack_idx(all_v, blk_v, None, nb=1)
        pltpu.async_copy(tq_hbm.at[blk_v.at[0]], rows_v.at[0], sem).wait()
        pltpu.sync_copy(rows_v.at[0], gq_o.at[pl.ds(off, CH)])
        return carry
    lax.fori_loop(0, n_t - nr * NB, tail, 0)


def _t0_body(ek_ref, src_ref, out_ref):
    out_ref[...] = ek_ref[...] * M + src_ref[...]


def _t0_gidx(ek, src):
    f = pl.pallas_call(
        _t0_body,
        out_shape=jax.ShapeDtypeStruct((E // 128, 128), jnp.int32),
    )
    return f(ek.reshape(E // 128, 128), src.reshape(E // 128, 128)).reshape(E)


BM = 1000


def _t1zr_body(sh0_ref, sh1_ref, sx0_ref, sx1_ref, cn0_ref, cn1_ref,
               w_ref, tv_ref):
    cnt = cn0_ref[0][:, 0:1] + cn1_ref[0][:, 0:1]
    recip = 1.0 / jnp.maximum(cnt, 1.0)
    vh = (sh0_ref[0] + sh1_ref[0]) * recip
    vx = (sx0_ref[0] + sx1_ref[0]) * recip
    w = w_ref[0, 0]
    tv_ref[0, 0] = (jnp.dot(vh, w[:H], preferred_element_type=_f32)
                    + jnp.dot(vx, w[H:], preferred_element_type=_f32))


def _t1zr(sums_h2, sums_x2, cnt2, Wzr):
    grid = (2, M // BM, K)
    ps = lambda which: pl.BlockSpec((1, BM, H), lambda zr, mb, k, w=which: (w, mb, 0))
    cs_ = ps
    f = pl.pallas_call(
        _t1zr_body,
        grid=grid,
        in_specs=[
            ps(0), ps(1), ps(0), ps(1), cs_(0), cs_(1),
            pl.BlockSpec((1, 1, 2 * H, H), lambda zr, mb, k: (zr, k, 0, 0)),
        ],
        out_specs=pl.BlockSpec((1, 1, BM, H), lambda zr, mb, k: (zr, k, mb, 0)),
        out_shape=jax.ShapeDtypeStruct((2, K, M, H), _f32),
    )
    return f(sums_h2, sums_h2, sums_x2, sums_x2, cnt2, cnt2, Wzr)


def _t1q_body(p0_ref, p1_ref, sx0_ref, sx1_ref, cn0_ref, cn1_ref,
              wq_ref, tvq_ref):
    cnt = cn0_ref[0][:, 0:1] + cn1_ref[0][:, 0:1]
    recip = 1.0 / jnp.maximum(cnt, 1.0)
    vrh = (p0_ref[0] + p1_ref[0]) * recip
    vx = (sx0_ref[0] + sx1_ref[0]) * recip
    wq = wq_ref[0]
    tvq_ref[0] = (jnp.dot(vrh, wq[:H], preferred_element_type=_f32)
                  + jnp.dot(vx, wq[H:], preferred_element_type=_f32))


def _t1q(sums_rh2, sums_x2, cnt2, Wq_c):
    grid = (M // BM, K)
    f = pl.pallas_call(
        _t1q_body,
        grid=grid,
        in_specs=[
            pl.BlockSpec((1, BM, H), lambda mb, k: (0, mb, 0)),
            pl.BlockSpec((1, BM, H), lambda mb, k: (1, mb, 0)),
            pl.BlockSpec((1, BM, H), lambda mb, k: (0, mb, 0)),
            pl.BlockSpec((1, BM, H), lambda mb, k: (1, mb, 0)),
            pl.BlockSpec((1, BM, H), lambda mb, k: (0, mb, 0)),
            pl.BlockSpec((1, BM, H), lambda mb, k: (1, mb, 0)),
            pl.BlockSpec((1, 2 * H, H), lambda mb, k: (k, 0, 0)),
        ],
        out_specs=pl.BlockSpec((1, BM, H), lambda mb, k: (k, mb, 0)),
        out_shape=jax.ShapeDtypeStruct((K, M, H), _f32),
    )
    return f(sums_rh2, sums_rh2, sums_x2, sums_x2, cnt2, cnt2, Wq_c)


BN = 2000


def _t2_body(h_ref, x_ref, gz_ref, gr_ref, wzl_ref, wrl_ref, wql_ref,
             bz_ref, br_ref, bq_ref, z_ref, rh_ref, qlin_ref):
    h = h_ref[...]
    x = x_ref[...]
    lin_z = (jnp.dot(h, wzl_ref[:H], preferred_element_type=_f32)
             + jnp.dot(x, wzl_ref[H:], preferred_element_type=_f32)
             + bz_ref[...])
    lin_r = (jnp.dot(h, wrl_ref[:H], preferred_element_type=_f32)
             + jnp.dot(x, wrl_ref[H:], preferred_element_type=_f32)
             + br_ref[...])
    z = jax.nn.sigmoid(gz_ref[0] + lin_z)
    r = jax.nn.sigmoid(gr_ref[0] + lin_r)
    rh = r * h
    qlin = (jnp.dot(rh, wql_ref[:H], preferred_element_type=_f32)
            + jnp.dot(x, wql_ref[H:], preferred_element_type=_f32)
            + bq_ref[...])
    z_ref[...] = z
    rh_ref[...] = rh
    qlin_ref[...] = qlin


def _t2(h_F, x_F, g2, Wz_l, bz_l, Wr_l, br_l, Wq_l, bq_l):
    grid = (N // BN,)
    row = pl.BlockSpec((BN, H), lambda i: (i, 0))
    gz_spec = pl.BlockSpec((1, BN, H), lambda i: (0, i, 0))
    gr_spec = pl.BlockSpec((1, BN, H), lambda i: (1, i, 0))
    wfull = pl.BlockSpec((2 * H, H), lambda i: (0, 0))
    bfull = pl.BlockSpec((1, H), lambda i: (0, 0))
    f = pl.pallas_call(
        _t2_body,
        grid=grid,
        in_specs=[row, row, gz_spec, gr_spec, wfull, wfull, wfull,
                  bfull, bfull, bfull],
        out_specs=[row, row, row],
        out_shape=[jax.ShapeDtypeStruct((N, H), _f32)] * 3,
    )
    return f(h_F, x_F, g2, g2, Wz_l, Wr_l, Wq_l,
             bz_l.reshape(1, H), br_l.reshape(1, H), bq_l.reshape(1, H))


def _tadd_body(a_ref, b_ref, o_ref):
    o_ref[0] = a_ref[0] + b_ref[0]


def _t_add(ab):
    grid = (M // BM,)
    f = pl.pallas_call(
        _tadd_body,
        grid=grid,
        in_specs=[pl.BlockSpec((1, BM, H), lambda i: (0, i, 0)),
                  pl.BlockSpec((1, BM, H), lambda i: (1, i, 0))],
        out_specs=pl.BlockSpec((1, BM, H), lambda i: (0, i, 0)),
        out_shape=jax.ShapeDtypeStruct((1, M, H), _f32),
    )
    return f(ab, ab)[0]


def _t3_body(h_ref, z_ref, qlin_ref, gq_ref, out_ref):
    z = z_ref[...]
    q = jnp.tanh(gq_ref[...] + qlin_ref[...])
    out_ref[...] = (1.0 - z) * h_ref[...] + z * q


def _t3(h_F, z, qlin, gq):
    grid = (N // BN,)
    row = pl.BlockSpec((BN, H), lambda i: (i, 0))
    f = pl.pallas_call(
        _t3_body,
        grid=grid,
        in_specs=[row, row, row, row],
        out_specs=row,
        out_shape=jax.ShapeDtypeStruct((N, H), _f32),
    )
    return f(h_F, z, qlin, gq)


def kernel(h_F, x_F, point2voxel, edge_index, edge_kernel,
           Wz_c, Wz_l, bz_l, Wr_c, Wr_l, br_l, Wq_c, Wq_l, bq_l):
    src = edge_index[0]
    dst = edge_index[1]
    z128 = jnp.zeros((MTL, H), _f32)
    ones128 = jnp.ones((CH, H), _f32)
    Wzr = jnp.stack([Wz_c, Wr_c])

    gidx = _t0_gidx(edge_kernel, src)

    sums_h2 = _sc_vox_ps(h_F, point2voxel, z128).reshape(NC, M, H)
    sums_x2 = _sc_vox_ps(x_F, point2voxel, z128).reshape(NC, M, H)
    cnt2 = _sc_cnt_ps(point2voxel, z128, ones128).reshape(NC, M, H)
    tv2 = _t1zr(sums_h2, sums_x2, cnt2, Wzr)
    ov2 = _sc_edge_zr(tv2.reshape(2 * KM, H), gidx, dst, z128)
    g2 = _sc_devox_zr(ov2, point2voxel).reshape(NC, N, H)
    z, rh, qlin = _t2(h_F, x_F, g2, Wz_l, bz_l, Wr_l, br_l, Wq_l, bq_l)

    sums_rh2 = _sc_vox_ps(rh, point2voxel, z128).reshape(NC, M, H)
    tv_q = _t1q(sums_rh2, sums_x2, cnt2, Wq_c)
    ovq2 = _sc_edge_q(tv_q.reshape(KM, H), gidx, dst, z128)
    ovq = _t_add(ovq2.reshape(NC, M, H))
    gq = _sc_devox_q(ovq, point2voxel)

    return _t3(h_F, z, qlin, gq)


# speedup vs baseline: 1.0092x; 1.0092x over previous
import functools

import jax
import jax.numpy as jnp
from jax import lax
from jax.experimental import pallas as pl
from jax.experimental.pallas import tpu as pltpu
from jax.experimental.pallas import tpu_sc as plsc

N = 100000
M = 10000
E = 160000
K = 27
H = 128
KM = K * M

NC = 2
NS = 16
CH = 80
NCH_P = N // CH
NCH_E = E // CH
MT0 = 624
MTL = M - (NS - 1) * MT0

_mesh = plsc.VectorSubcoreMesh(
    core_axis_name="c", subcore_axis_name="s", num_cores=NC, num_subcores=NS)

_f32 = jnp.float32


def _cs():
    return lax.axis_index("c"), lax.axis_index("s")


def _tile_copy(s, src_fn, dst_fn):
    base = pl.multiple_of(s * MT0, 8)
    last = (NS - 1) * MT0

    @pl.when(s < NS - 1)
    def _():
        pltpu.sync_copy(src_fn(base, MT0), dst_fn(base, MT0))

    @pl.when(s == NS - 1)
    def _():
        pltpu.sync_copy(src_fn(last, MTL), dst_fn(last, MTL))


NB = 4
NBD = 10


def _unpack_idx(all_v, blk_v, delta, nb=NB):
    for b in range(nb):
        for u in range(CH // 16):
            v = all_v[pl.ds(b * CH + u * 16, 16)]
            if delta is not None:
                v = v + delta
            blk_v[b, pl.ds(u * 16, 16)] = v


def _tile_span(total, s):
    q0, rem = divmod(total, NS)
    base = s * q0 + jnp.minimum(s, rem)
    n = q0 + jnp.where(s < rem, 1, 0)
    return base, n


@functools.partial(
    pl.kernel,
    out_type=jax.ShapeDtypeStruct((NC * M, H), _f32),
    mesh=_mesh,
    scratch_types=[
        pltpu.VMEM_SHARED((M, H), _f32),
        pltpu.VMEM((NB * CH,), jnp.int32),
        pltpu.VMEM((NB, CH), jnp.int32),
        pltpu.VMEM((NB, CH, H), _f32),
        pltpu.SemaphoreType.DMA,
        pltpu.SemaphoreType.DMA,
    ],
)
def _sc_vox_ps(data_hbm, p2v_hbm, z128,
               sums_o, acc, all_v, blk_v, rows_v, sem, sem2):
    c, s = _cs()
    _tile_copy(s, lambda b, n: z128.at[pl.ds(0, n)],
               lambda b, n: acc.at[pl.ds(b, n)])
    plsc.subcore_barrier()

    half = NCH_P // NC
    base, n_t = _tile_span(half, s)
    base = base + c * half
    nr = n_t // NB

    def rnd(r, carry):
        cb = base + r * NB
        off = pl.multiple_of(cb * CH, 8)
        pltpu.sync_copy(p2v_hbm.at[pl.ds(off, NB * CH)], all_v)
        _unpack_idx(all_v, blk_v, None)
        gd = [pltpu.async_copy(
            data_hbm.at[pl.ds(pl.multiple_of((cb + b) * CH, 8), CH)],
            rows_v.at[b], sem) for b in range(NB)]
        for d in gd:
            d.wait()
        sd = [pltpu.async_copy(rows_v.at[b], acc.at[blk_v.at[b]], sem2,
                               add=True) for b in range(NB)]
        for d in sd:
            d.wait()
        return carry
    lax.fori_loop(0, nr, rnd, 0)

    def tail(i, carry):
        cb = base + nr * NB + i
        off = pl.multiple_of(cb * CH, 8)
        pltpu.sync_copy(p2v_hbm.at[pl.ds(off, CH)], all_v.at[pl.ds(0, CH)])
        _unpack_idx(all_v, blk_v, None, nb=1)
        pltpu.sync_copy(data_hbm.at[pl.ds(off, CH)], rows_v.at[0])
        pltpu.sync_copy(rows_v.at[0], acc.at[blk_v.at[0]], add=True)
        return carry
    lax.fori_loop(0, n_t - nr * NB, tail, 0)

    plsc.subcore_barrier()
    cm = pl.multiple_of(c * M, 8)
    _tile_copy(s, lambda b, n: acc.at[pl.ds(b, n)],
               lambda b, n: sums_o.at[pl.ds(cm + b, n)])


@functools.partial(
    pl.kernel,
    out_type=jax.ShapeDtypeStruct((NC * M, H), _f32),
    mesh=_mesh,
    scratch_types=[
        pltpu.VMEM_SHARED((M, H), _f32),
        pltpu.VMEM((NB * CH,), jnp.int32),
        pltpu.VMEM((NB, CH), jnp.int32),
        pltpu.VMEM((CH, H), _f32),
        pltpu.SemaphoreType.DMA,
    ],
)
def _sc_cnt_ps(p2v_hbm, z128, ones128, cnt_o, acc, all_v, blk_v, ones_v,
               sem2):
    c, s = _cs()
    _tile_copy(s, lambda b, n: z128.at[pl.ds(0, n)],
               lambda b, n: acc.at[pl.ds(b, n)])
    pltpu.sync_copy(ones128, ones_v)
    plsc.subcore_barrier()

    half = NCH_P // NC
    base, n_t = _tile_span(half, s)
    base = base + c * half
    nr = n_t // NB

    def rnd(r, carry):
        cb = base + r * NB
        off = pl.multiple_of(cb * CH, 8)
        pltpu.sync_copy(p2v_hbm.at[pl.ds(off, NB * CH)], all_v)
        _unpack_idx(all_v, blk_v, None)
        sd = [pltpu.async_copy(ones_v, acc.at[blk_v.at[b]], sem2,
                               add=True) for b in range(NB)]
        for d in sd:
            d.wait()
        return carry
    lax.fori_loop(0, nr, rnd, 0)

    def tail(i, carry):
        cb = base + nr * NB + i
        off = pl.multiple_of(cb * CH, 8)
        pltpu.sync_copy(p2v_hbm.at[pl.ds(off, CH)], all_v.at[pl.ds(0, CH)])
        _unpack_idx(all_v, blk_v, None, nb=1)
        pltpu.sync_copy(ones_v, acc.at[blk_v.at[0]], add=True)
        return carry
    lax.fori_loop(0, n_t - nr * NB, tail, 0)

    plsc.subcore_barrier()
    cm = pl.multiple_of(c * M, 8)
    _tile_copy(s, lambda b, n: acc.at[pl.ds(b, n)],
               lambda b, n: cnt_o.at[pl.ds(cm + b, n)])


@functools.partial(
    pl.kernel,
    out_type=jax.ShapeDtypeStruct((NC * M, H), _f32),
    mesh=_mesh,
    scratch_types=[
        pltpu.VMEM_SHARED((M, H), _f32),
        pltpu.VMEM((NB * CH,), jnp.int32),
        pltpu.VMEM((NB * CH,), jnp.int32),
        pltpu.VMEM((NB, CH), jnp.int32),
        pltpu.VMEM((NB, CH), jnp.int32),
        pltpu.VMEM((NB, CH, H), _f32),
        pltpu.SemaphoreType.DMA,
        pltpu.SemaphoreType.DMA,
    ],
)
def _sc_edge_zr(tv2_hbm, gidx_hbm, dst_hbm, z128, out2_o,
                acc, gi_all, di_all, gi_blk, di_blk, rows_v, sem, sem2):
    c, s = _cs()
    _tile_copy(s, lambda b, n: z128.at[pl.ds(0, n)],
               lambda b, n: acc.at[pl.ds(b, n)])
    plsc.subcore_barrier()

    base, n_t = _tile_span(NCH_E, s)
    nr = n_t // NB
    delta = c * KM

    def rnd(r, carry):
        cb = base + r * NB
        off = pl.multiple_of(cb * CH, 8)
        pltpu.sync_copy(gidx_hbm.at[pl.ds(off, NB * CH)], gi_all)
        pltpu.sync_copy(dst_hbm.at[pl.ds(off, NB * CH)], di_all)
        _unpack_idx(gi_all, gi_blk, delta)
        _unpack_idx(di_all, di_blk, None)
        gd = [pltpu.async_copy(tv2_hbm.at[gi_blk.at[b]], rows_v.at[b], sem)
              for b in range(NB)]
        for d in gd:
            d.wait()
        sd = [pltpu.async_copy(rows_v.at[b], acc.at[di_blk.at[b]], sem2,
                               add=True) for b in range(NB)]
        for d in sd:
            d.wait()
        return carry
    lax.fori_loop(0, nr, rnd, 0)

    def tail(i, carry):
        cb = base + nr * NB + i
        off = pl.multiple_of(cb * CH, 8)
        pltpu.sync_copy(gidx_hbm.at[pl.ds(off, CH)], gi_all.at[pl.ds(0, CH)])
        pltpu.sync_copy(dst_hbm.at[pl.ds(off, CH)], di_all.at[pl.ds(0, CH)])
        _unpack_idx(gi_all, gi_blk, delta, nb=1)
        _unpack_idx(di_all, di_blk, None, nb=1)
        pltpu.async_copy(tv2_hbm.at[gi_blk.at[0]], rows_v.at[0], sem).wait()
        pltpu.sync_copy(rows_v.at[0], acc.at[di_blk.at[0]], add=True)
        return carry
    lax.fori_loop(0, n_t - nr * NB, tail, 0)

    plsc.subcore_barrier()
    cm = pl.multiple_of(c * M, 8)
    _tile_copy(s, lambda b, n: acc.at[pl.ds(b, n)],
               lambda b, n: out2_o.at[pl.ds(cm + b, n)])


@functools.partial(
    pl.kernel,
    out_type=jax.ShapeDtypeStruct((NC * N, H), _f32),
    mesh=_mesh,
    scratch_types=[
        pltpu.VMEM((NBD * CH,), jnp.int32),
        pltpu.VMEM((NBD, CH), jnp.int32),
        pltpu.VMEM((NBD, CH, H), _f32),
        pltpu.SemaphoreType.DMA,
        pltpu.SemaphoreType.DMA,
    ],
)
def _sc_devox_zr(ov2_hbm, p2v_hbm, g2_o, all_v, blk_v, rows_v, sem, sem2):
    c, s = _cs()
    base, n_t = _tile_span(NCH_P, s)
    nr = n_t // NBD
    delta = c * M

    def rnd(r, carry):
        cb = base + r * NBD
        off = pl.multiple_of(cb * CH, 8)
        pltpu.sync_copy(p2v_hbm.at[pl.ds(off, NBD * CH)], all_v)
        _unpack_idx(all_v, blk_v, delta, nb=NBD)
        gd = [pltpu.async_copy(ov2_hbm.at[blk_v.at[b]], rows_v.at[b], sem)
              for b in range(NBD)]
        for d in gd:
            d.wait()
        cn = pl.multiple_of(c * N + off, 8)
        sd = [pltpu.async_copy(
            rows_v.at[b], g2_o.at[pl.ds(cn + b * CH, CH)], sem2)
            for b in range(NBD)]
        for d in sd:
            d.wait()
        return carry
    lax.fori_loop(0, nr, rnd, 0)

    def tail(i, carry):
        cb = base + nr * NBD + i
        off = pl.multiple_of(cb * CH, 8)
        pltpu.sync_copy(p2v_hbm.at[pl.ds(off, CH)], all_v.at[pl.ds(0, CH)])
        _unpack_idx(all_v, blk_v, delta, nb=1)
        pltpu.async_copy(ov2_hbm.at[blk_v.at[0]], rows_v.at[0], sem).wait()
        cn = pl.multiple_of(c * N + off, 8)
        pltpu.sync_copy(rows_v.at[0], g2_o.at[pl.ds(cn, CH)])
        return carry
    lax.fori_loop(0, n_t - nr * NBD, tail, 0)


@functools.partial(
    pl.kernel,
    out_type=jax.ShapeDtypeStruct((NC * M, H), _f32),
    mesh=_mesh,
    scratch_types=[
        pltpu.VMEM_SHARED((M, H), _f32),
        pltpu.VMEM((NB * CH,), jnp.int32),
        pltpu.VMEM((NB * CH,), jnp.int32),
        pltpu.VMEM((NB, CH), jnp.int32),
        pltpu.VMEM((NB, CH), jnp.int32),
        pltpu.VMEM((NB, CH, H), _f32),
        pltpu.SemaphoreType.DMA,
        pltpu.SemaphoreType.DMA,
    ],
)
def _sc_edge_q(tvq_hbm, gidx_hbm, dst_hbm, z128, out2_o,
               acc, gi_all, di_all, gi_blk, di_blk, rows_v, sem, sem2):
    c, s = _cs()
    _tile_copy(s, lambda b, n: z128.at[pl.ds(0, n)],
               lambda b, n: acc.at[pl.ds(b, n)])
    plsc.subcore_barrier()

    half = NCH_E // NC
    base, n_t = _tile_span(half, s)
    base = base + c * half
    nr = n_t // NB

    def rnd(r, carry):
        cb = base + r * NB
        off = pl.multiple_of(cb * CH, 8)
        pltpu.sync_copy(gidx_hbm.at[pl.ds(off, NB * CH)], gi_all)
        pltpu.sync_copy(dst_hbm.at[pl.ds(off, NB * CH)], di_all)
        _unpack_idx(gi_all, gi_blk, None)
        _unpack_idx(di_all, di_blk, None)
        gd = [pltpu.async_copy(tvq_hbm.at[gi_blk.at[b]], rows_v.at[b], sem)
              for b in range(NB)]
        for d in gd:
            d.wait()
        sd = [pltpu.async_copy(rows_v.at[b], acc.at[di_blk.at[b]], sem2,
                               add=True) for b in range(NB)]
        for d in sd:
            d.wait()
        return carry
    lax.fori_loop(0, nr, rnd, 0)

    def tail(i, carry):
        cb = base + nr * NB + i
        off = pl.multiple_of(cb * CH, 8)
        pltpu.sync_copy(gidx_hbm.at[pl.ds(off, CH)], gi_all.at[pl.ds(0, CH)])
        pltpu.sync_copy(dst_hbm.at[pl.ds(off, CH)], di_all.at[pl.ds(0, CH)])
        _unpack_idx(gi_all, gi_blk, None, nb=1)
        _unpack_idx(di_all, di_blk, None, nb=1)
        pltpu.async_copy(tvq_hbm.at[gi_blk.at[0]], rows_v.at[0], sem).wait()
        pltpu.sync_copy(rows_v.at[0], acc.at[di_blk.at[0]], add=True)
        return carry
    lax.fori_loop(0, n_t - nr * NB, tail, 0)

    plsc.subcore_barrier()
    cm = pl.multiple_of(c * M, 8)
    _tile_copy(s, lambda b, n: acc.at[pl.ds(b, n)],
               lambda b, n: out2_o.at[pl.ds(cm + b, n)])


@functools.partial(
    pl.kernel,
    out_type=jax.ShapeDtypeStruct((N, H), _f32),
    mesh=_mesh,
    scratch_types=[
        pltpu.VMEM((NBD * CH,), jnp.int32),
        pltpu.VMEM((NBD, CH), jnp.int32),
        pltpu.VMEM((NBD, CH, H), _f32),
        pltpu.SemaphoreType.DMA,
        pltpu.SemaphoreType.DMA,
    ],
)
def _sc_devox_q(tq_hbm, p2v_hbm, gq_o, all_v, blk_v, rows_v, sem, sem2):
    c, s = _cs()
    half = NCH_P // NC
    base, n_t = _tile_span(half, s)
    base = base + c * half
    nr = n_t // NBD

    def rnd(r, carry):
        cb = base + r * NBD
        off = pl.multiple_of(cb * CH, 8)
        pltpu.sync_copy(p2v_hbm.at[pl.ds(off, NBD * CH)], all_v)
        _unpack_idx(all_v, blk_v, None, nb=NBD)
        gd = [pltpu.async_copy(tq_hbm.at[blk_v.at[b]], rows_v.at[b], sem)
              for b in range(NBD)]
        for d in gd:
            d.wait()
        sd = [pltpu.async_copy(
            rows_v.at[b], gq_o.at[pl.ds(pl.multiple_of(off + b * CH, 8), CH)],
            sem2) for b in range(NBD)]
        for d in sd:
            d.wait()
        return carry
    lax.fori_loop(0, nr, rnd, 0)

    def tail(i, carry):
        cb = base + nr * NBD + i
        off = pl.multiple_of(cb * CH, 8)
        pltpu.sync_copy(p2v_hbm.at[pl.ds(off, CH)], all_v.at[pl.ds(0, CH)])
        _unpack_idx(all_v, blk_v, None, nb=1)
        pltpu.async_copy(tq_hbm.at[blk_v.at[0]], rows_v.at[0], sem).wait()
        pltpu.sync_copy(rows_v.at[0], gq_o.at[pl.ds(off, CH)])
        return carry
    lax.fori_loop(0, n_t - nr * NBD, tail, 0)


def _t0_body(ek_ref, src_ref, out_ref):
    out_ref[...] = ek_ref[...] * M + src_ref[...]


def _t0_gidx(ek, src):
    f = pl.pallas_call(
        _t0_body,
        out_shape=jax.ShapeDtypeStruct((E // 128, 128), jnp.int32),
    )
    return f(ek.reshape(E // 128, 128), src.reshape(E // 128, 128)).reshape(E)


BM = 1000


def _t1zr_body(sh0_ref, sh1_ref, sx0_ref, sx1_ref, cn0_ref, cn1_ref,
               w_ref, tv_ref):
    cnt = cn0_ref[0][:, 0:1] + cn1_ref[0][:, 0:1]
    recip = 1.0 / jnp.maximum(cnt, 1.0)
    vh = (sh0_ref[0] + sh1_ref[0]) * recip
    vx = (sx0_ref[0] + sx1_ref[0]) * recip
    w = w_ref[0, 0]
    tv_ref[0, 0] = (jnp.dot(vh, w[:H], preferred_element_type=_f32)
                    + jnp.dot(vx, w[H:], preferred_element_type=_f32))


def _t1zr(sums_h2, sums_x2, cnt2, Wzr):
    grid = (2, M // BM, K)
    ps = lambda which: pl.BlockSpec((1, BM, H), lambda zr, mb, k, w=which: (w, mb, 0))
    cs_ = ps
    f = pl.pallas_call(
        _t1zr_body,
        grid=grid,
        in_specs=[
            ps(0), ps(1), ps(0), ps(1), cs_(0), cs_(1),
            pl.BlockSpec((1, 1, 2 * H, H), lambda zr, mb, k: (zr, k, 0, 0)),
        ],
        out_specs=pl.BlockSpec((1, 1, BM, H), lambda zr, mb, k: (zr, k, mb, 0)),
        out_shape=jax.ShapeDtypeStruct((2, K, M, H), _f32),
    )
    return f(sums_h2, sums_h2, sums_x2, sums_x2, cnt2, cnt2, Wzr)


def _t1q_body(p0_ref, p1_ref, sx0_ref, sx1_ref, cn0_ref, cn1_ref,
              wq_ref, tvq_ref):
    cnt = cn0_ref[0][:, 0:1] + cn1_ref[0][:, 0:1]
    recip = 1.0 / jnp.maximum(cnt, 1.0)
    vrh = (p0_ref[0] + p1_ref[0]) * recip
    vx = (sx0_ref[0] + sx1_ref[0]) * recip
    wq = wq_ref[0]
    tvq_ref[0] = (jnp.dot(vrh, wq[:H], preferred_element_type=_f32)
                  + jnp.dot(vx, wq[H:], preferred_element_type=_f32))


def _t1q(sums_rh2, sums_x2, cnt2, Wq_c):
    grid = (M // BM, K)
    f = pl.pallas_call(
        _t1q_body,
        grid=grid,
        in_specs=[
            pl.BlockSpec((1, BM, H), lambda mb, k: (0, mb, 0)),
            pl.BlockSpec((1, BM, H), lambda mb, k: (1, mb, 0)),
            pl.BlockSpec((1, BM, H), lambda mb, k: (0, mb, 0)),
            pl.BlockSpec((1, BM, H), lambda mb, k: (1, mb, 0)),
            pl.BlockSpec((1, BM, H), lambda mb, k: (0, mb, 0)),
            pl.BlockSpec((1, BM, H), lambda mb, k: (1, mb, 0)),
            pl.BlockSpec((1, 2 * H, H), lambda mb, k: (k, 0, 0)),
        ],
        out_specs=pl.BlockSpec((1, BM, H), lambda mb, k: (k, mb, 0)),
        out_shape=jax.ShapeDtypeStruct((K, M, H), _f32),
    )
    return f(sums_rh2, sums_rh2, sums_x2, sums_x2, cnt2, cnt2, Wq_c)


BN = 2000


def _t2_body(h_ref, x_ref, gz_ref, gr_ref, wzl_ref, wrl_ref, wql_ref,
             bz_ref, br_ref, bq_ref, z_ref, rh_ref, qlin_ref):
    h = h_ref[...]
    x = x_ref[...]
    lin_z = (jnp.dot(h, wzl_ref[:H], preferred_element_type=_f32)
             + jnp.dot(x, wzl_ref[H:], preferred_element_type=_f32)
             + bz_ref[...])
    lin_r = (jnp.dot(h, wrl_ref[:H], preferred_element_type=_f32)
             + jnp.dot(x, wrl_ref[H:], preferred_element_type=_f32)
             + br_ref[...])
    z = jax.nn.sigmoid(gz_ref[0] + lin_z)
    r = jax.nn.sigmoid(gr_ref[0] + lin_r)
    rh = r * h
    qlin = (jnp.dot(rh, wql_ref[:H], preferred_element_type=_f32)
            + jnp.dot(x, wql_ref[H:], preferred_element_type=_f32)
            + bq_ref[...])
    z_ref[...] = z
    rh_ref[...] = rh
    qlin_ref[...] = qlin


def _t2(h_F, x_F, g2, Wz_l, bz_l, Wr_l, br_l, Wq_l, bq_l):
    grid = (N // BN,)
    row = pl.BlockSpec((BN, H), lambda i: (i, 0))
    gz_spec = pl.BlockSpec((1, BN, H), lambda i: (0, i, 0))
    gr_spec = pl.BlockSpec((1, BN, H), lambda i: (1, i, 0))
    wfull = pl.BlockSpec((2 * H, H), lambda i: (0, 0))
    bfull = pl.BlockSpec((1, H), lambda i: (0, 0))
    f = pl.pallas_call(
        _t2_body,
        grid=grid,
        in_specs=[row, row, gz_spec, gr_spec, wfull, wfull, wfull,
                  bfull, bfull, bfull],
        out_specs=[row, row, row],
        out_shape=[jax.ShapeDtypeStruct((N, H), _f32)] * 3,
    )
    return f(h_F, x_F, g2, g2, Wz_l, Wr_l, Wq_l,
             bz_l.reshape(1, H), br_l.reshape(1, H), bq_l.reshape(1, H))


def _tadd_body(a_ref, b_ref, o_ref):
    o_ref[0] = a_ref[0] + b_ref[0]


def _t_add(ab):
    grid = (M // BM,)
    f = pl.pallas_call(
        _tadd_body,
        grid=grid,
        in_specs=[pl.BlockSpec((1, BM, H), lambda i: (0, i, 0)),
                  pl.BlockSpec((1, BM, H), lambda i: (1, i, 0))],
        out_specs=pl.BlockSpec((1, BM, H), lambda i: (0, i, 0)),
        out_shape=jax.ShapeDtypeStruct((1, M, H), _f32),
    )
    return f(ab, ab)[0]


def _t3_body(h_ref, z_ref, qlin_ref, gq_ref, out_ref):
    z = z_ref[...]
    q = jnp.tanh(gq_ref[...] + qlin_ref[...])
    out_ref[...] = (1.0 - z) * h_ref[...] + z * q


def _t3(h_F, z, qlin, gq):
    grid = (N // BN,)
    row = pl.BlockSpec((BN, H), lambda i: (i, 0))
    f = pl.pallas_call(
        _t3_body,
        grid=grid,
        in_specs=[row, row, row, row],
        out_specs=row,
        out_shape=jax.ShapeDtypeStruct((N, H), _f32),
    )
    return f(h_F, z, qlin, gq)


def kernel(h_F, x_F, point2voxel, edge_index, edge_kernel,
           Wz_c, Wz_l, bz_l, Wr_c, Wr_l, br_l, Wq_c, Wq_l, bq_l):
    src = edge_index[0]
    dst = edge_index[1]
    z128 = jnp.zeros((MTL, H), _f32)
    ones128 = jnp.ones((CH, H), _f32)
    Wzr = jnp.stack([Wz_c, Wr_c])

    gidx = _t0_gidx(edge_kernel, src)

    sums_h2 = _sc_vox_ps(h_F, point2voxel, z128).reshape(NC, M, H)
    sums_x2 = _sc_vox_ps(x_F, point2voxel, z128).reshape(NC, M, H)
    cnt2 = _sc_cnt_ps(point2voxel, z128, ones128).reshape(NC, M, H)
    tv2 = _t1zr(sums_h2, sums_x2, cnt2, Wzr)
    ov2 = _sc_edge_zr(tv2.reshape(2 * KM, H), gidx, dst, z128)
    g2 = _sc_devox_zr(ov2, point2voxel).reshape(NC, N, H)
    z, rh, qlin = _t2(h_F, x_F, g2, Wz_l, bz_l, Wr_l, br_l, Wq_l, bq_l)

    sums_rh2 = _sc_vox_ps(rh, point2voxel, z128).reshape(NC, M, H)
    tv_q = _t1q(sums_rh2, sums_x2, cnt2, Wq_c)
    ovq2 = _sc_edge_q(tv_q.reshape(KM, H), gidx, dst, z128)
    ovq = _t_add(ovq2.reshape(NC, M, H))
    gq = _sc_devox_q(ovq, point2voxel)

    return _t3(h_F, z, qlin, gq)
